# baseline (speedup 1.0000x reference)
import jax
import jax.numpy as jnp
from jax import lax
from jax.experimental import pallas as pl
from jax.experimental.pallas import tpu as pltpu
from jax.experimental.pallas import tpu_sc as plsc

_N = 10000
_E = 320000
_NFEAT = 128
_NHID = 64
_G = 16
_EPS = 1e-5

_NC = 2
_NS = 16
_NW = _NC * _NS
_W = 128
_WPW = 80
_NWIN = _NW * _WPW
_EPAD = _NWIN * _W
_NPAD = 10240
_RPT = _NPAD // _NS


_K = 2
_NPH = 2
_WPP = _WPW // _NPH
_NG = _WPP // _K


def _segsum_body(h_hbm, src_hbm, dst_hbm, out_hbm,
                 src_buf, dst_buf, rows_v, agg_sh, h_sh,
                 sem_g0, sem_g1, sem_s0, sem_s1):
    c = lax.axis_index("c")
    s = lax.axis_index("s")
    wid = c * _NS + s
    r0 = pl.multiple_of(s * _RPT, 8)
    sem_g = (sem_g0, sem_g1)
    sem_s = (sem_s0, sem_s1)

    def _zero_row(r, _):
        for j in range(_NHID // 16):
            rows_v[0, r, pl.ds(j * 16, 16)] = jnp.zeros((16,), jnp.float32)
        return 0
    lax.fori_loop(0, _W, _zero_row, 0)
    for k in range(_RPT // _W):
        pltpu.sync_copy(rows_v.at[0], agg_sh.at[pl.ds(r0 + k * _W, _W)])

    @pl.when(s < _NS - 1)
    def _():
        hb = pl.multiple_of(s * _RPT, 8)
        pltpu.sync_copy(h_hbm.at[pl.ds(hb, _RPT)], h_sh.at[pl.ds(hb, _RPT)])

    @pl.when(s == _NS - 1)
    def _():
        hb = (_NS - 1) * _RPT
        rem = _N - hb
        pltpu.sync_copy(h_hbm.at[pl.ds(hb, rem)], h_sh.at[pl.ds(hb, rem)])

    def _fire_gathers(g, p):
        for b in range(_K):
            pltpu.async_copy(h_sh.at[src_buf.at[g * _K + b]],
                             rows_v.at[p * _K + b], sem_g[p])

    def _drain_gathers(p):
        for b in range(_K):
            pltpu.make_async_copy(h_hbm.at[src_buf.at[0]],
                                  rows_v.at[p * _K + b], sem_g[p]).wait()

    def _fire_scatters(g, p):
        for b in range(_K):
            pltpu.async_copy(rows_v.at[p * _K + b],
                             agg_sh.at[dst_buf.at[g * _K + b]], sem_s[p],
                             add=True)

    def _drain_scatters(p):
        for b in range(_K):
            pltpu.make_async_copy(rows_v.at[p * _K + b],
                                  agg_sh.at[dst_buf.at[0]], sem_s[p]).wait()

    for ph in range(_NPH):
        w0 = pl.multiple_of(wid * _WPW + ph * _WPP, 8)
        pltpu.sync_copy(src_hbm.at[pl.ds(w0, _WPP)], src_buf)
        pltpu.sync_copy(dst_hbm.at[pl.ds(w0, _WPP)], dst_buf)
        if ph == 0:
            plsc.subcore_barrier()

        _fire_gathers(0, 0)

        def _pair(i, _):
            g = 2 * i
            _drain_gathers(0)

            @pl.when(i > 0)
            def _():
                _drain_scatters(1)

            _fire_gathers(g + 1, 1)
            _fire_scatters(g, 0)
            _drain_gathers(1)
            _drain_scatters(0)

            @pl.when(g + 2 < _NG)
            def _():
                _fire_gathers(g + 2, 0)

            _fire_scatters(g + 1, 1)
            return 0

        lax.fori_loop(0, _NG // 2, _pair, 0)
        _drain_scatters(1)

    plsc.subcore_barrier()
    for k in range(_RPT // _W):
        pltpu.sync_copy(agg_sh.at[pl.ds(r0 + k * _W, _W)], rows_v.at[0])
        pltpu.sync_copy(rows_v.at[0], out_hbm.at[c, pl.ds(r0 + k * _W, _W)])


@jax.jit
def _segsum(h, src2d, dst2d):
    mesh = plsc.VectorSubcoreMesh(core_axis_name="c", subcore_axis_name="s")
    kfn = pl.kernel(
        _segsum_body,
        out_type=jax.ShapeDtypeStruct((_NC, _NPAD, _NHID), jnp.float32),
        mesh=mesh,
        scratch_types=[
            pltpu.VMEM((_WPP, _W), jnp.int32),
            pltpu.VMEM((_WPP, _W), jnp.int32),
            pltpu.VMEM((2 * _K, _W, _NHID), jnp.float32),
            pltpu.VMEM_SHARED((_NPAD, _NHID), jnp.float32),
            pltpu.VMEM_SHARED((_N, _NHID), jnp.float32),
            pltpu.SemaphoreType.DMA,
            pltpu.SemaphoreType.DMA,
            pltpu.SemaphoreType.DMA,
            pltpu.SemaphoreType.DMA,
        ],
        compiler_params=pltpu.CompilerParams(use_tc_tiling_on_sc=False),
    )
    return kfn(h, src2d, dst2d)


def _pool(h, batch2d):
    onehot = (batch2d == lax.broadcasted_iota(jnp.int32, (_N, _G), 1))
    onehot = onehot.astype(jnp.float32)
    s = lax.dot_general(onehot, h, (((0,), (0,)), ((), ())),
                        preferred_element_type=jnp.float32)
    cnt = jnp.sum(onehot, axis=0)[:, None]
    return s / jnp.maximum(cnt, 1.0)


def _bn(h, g, b):
    mu = jnp.mean(h, axis=0)
    var = jnp.mean((h - mu) ** 2, axis=0)
    return g * (h - mu) * lax.rsqrt(var + _EPS) + b


def _prologue_body(x_ref, w_ref, b_ref, g_ref, be_ref, batch_ref,
                   h_ref, emb_ref):
    h = jnp.dot(x_ref[...], w_ref[...], preferred_element_type=jnp.float32)
    h = _bn(h + b_ref[...], g_ref[...], be_ref[...])
    h_ref[...] = h
    emb_ref[...] = _pool(h, batch_ref[...])


@jax.jit
def _prologue(x, w, b, g, be, batch2d):
    return pl.pallas_call(
        _prologue_body,
        out_shape=(jax.ShapeDtypeStruct((_N, _NHID), jnp.float32),
                   jax.ShapeDtypeStruct((_G, _NHID), jnp.float32)),
    )(x, w, b, g, be, batch2d)


def _layer_body(h_ref, agg_ref, wa_ref, ba_ref, wb_ref, bb_ref,
                g_ref, be_ref, batch_ref, ho_ref, emb_ref):
    z = h_ref[...] + agg_ref[0, :_N, :] + agg_ref[1, :_N, :]
    m = jnp.dot(z, wa_ref[...], preferred_element_type=jnp.float32)
    m = jnp.maximum(m + ba_ref[...], 0.0)
    m = jnp.dot(m, wb_ref[...], preferred_element_type=jnp.float32)
    m = m + bb_ref[...]
    h = jnp.maximum(_bn(m, g_ref[...], be_ref[...]), 0.0)
    ho_ref[...] = h
    emb_ref[...] = _pool(h, batch_ref[...])


@jax.jit
def _layer(h, aggs, wa, ba, wb, bb, g, be, batch2d):
    return pl.pallas_call(
        _layer_body,
        out_shape=(jax.ShapeDtypeStruct((_N, _NHID), jnp.float32),
                   jax.ShapeDtypeStruct((_G, _NHID), jnp.float32)),
    )(h, aggs, wa, ba, wb, bb, g, be, batch2d)


def kernel(x, params, edge_index, batch):
    npad_e = _EPAD - _E
    pad_iota = jnp.arange(npad_e, dtype=jnp.int32)
    src2d = jnp.concatenate([edge_index[0], pad_iota % _N]).reshape(_NWIN, _W)
    dst2d = jnp.concatenate(
        [edge_index[1], _N + pad_iota % (_NPAD - _N)]).reshape(_NWIN, _W)
    batch2d = batch[:, None]
    p = params
    h, emb = _prologue(x, p['W0'], p['b0'][None, :], p['g0'][None, :],
                       p['be0'][None, :], batch2d)
    embeds = [emb]
    for L in p['layers']:
        aggs = _segsum(h, src2d, dst2d)
        h, emb = _layer(h, aggs, L['Wa'], L['ba'][None, :], L['Wb'],
                        L['bb'][None, :], L['g'][None, :], L['be'][None, :],
                        batch2d)
        embeds.append(emb)
    return jnp.stack(embeds)


# speedup vs baseline: 1.4537x; 1.4537x over previous
import jax
import jax.numpy as jnp
from jax import lax
from jax.experimental import pallas as pl
from jax.experimental.pallas import tpu as pltpu
from jax.experimental.pallas import tpu_sc as plsc

_N = 10000
_E = 320000
_NFEAT = 128
_NHID = 64
_G = 16
_EPS = 1e-5

_NC = 2
_NS = 16
_NW = _NC * _NS
_W = 128
_WPW = 80
_NWIN = _NW * _WPW
_EPAD = _NWIN * _W
_NPAD = 10240
_RPT = _NPAD // _NS

_NP2 = _N // 2


_K = 4
_NG = _WPW // _K


def _segsum_body(h_hbm, edges_hbm, out_hbm,
                 src_buf, dst_buf, rows_v, agg_sh,
                 sem_g0, sem_g1, sem_s0, sem_s1):
    c = lax.axis_index("c")
    s = lax.axis_index("s")
    wid = c * _NS + s
    r0 = pl.multiple_of(s * _RPT, 8)
    sem_g = (sem_g0, sem_g1)
    sem_s = (sem_s0, sem_s1)

    def _zero_row(r, _):
        for j in range(_NHID // 16):
            rows_v[0, r, pl.ds(j * 16, 16)] = jnp.zeros((16,), jnp.float32)
        return 0
    lax.fori_loop(0, _W, _zero_row, 0)
    for k in range(_RPT // _W):
        pltpu.sync_copy(rows_v.at[0], agg_sh.at[pl.ds(r0 + k * _W, _W)])

    w0 = pl.multiple_of(wid * _WPW, 8)
    pltpu.sync_copy(edges_hbm.at[0, pl.ds(w0, _WPW)], src_buf)
    pltpu.sync_copy(edges_hbm.at[1, pl.ds(w0, _WPW)], dst_buf)
    plsc.subcore_barrier()

    def _fire_gathers(g, p):
        for b in range(_K):
            pltpu.async_copy(h_hbm.at[src_buf.at[g * _K + b]],
                             rows_v.at[p * _K + b], sem_g[p])

    def _drain_gathers(p):
        for b in range(_K):
            pltpu.make_async_copy(h_hbm.at[src_buf.at[0]],
                                  rows_v.at[p * _K + b], sem_g[p]).wait()

    def _fire_scatters(g, p):
        for b in range(_K):
            pltpu.async_copy(rows_v.at[p * _K + b],
                             agg_sh.at[dst_buf.at[g * _K + b]], sem_s[p],
                             add=True)

    def _drain_scatters(p):
        for b in range(_K):
            pltpu.make_async_copy(rows_v.at[p * _K + b],
                                  agg_sh.at[dst_buf.at[0]], sem_s[p]).wait()

    _fire_gathers(0, 0)

    def _pair(i, _):
        g = 2 * i
        _drain_gathers(0)

        @pl.when(i > 0)
        def _():
            _drain_scatters(1)

        _fire_gathers(g + 1, 1)
        _fire_scatters(g, 0)
        _drain_gathers(1)
        _drain_scatters(0)

        @pl.when(g + 2 < _NG)
        def _():
            _fire_gathers(g + 2, 0)

        _fire_scatters(g + 1, 1)
        return 0

    lax.fori_loop(0, _NG // 2, _pair, 0)
    _drain_scatters(1)

    plsc.subcore_barrier()
    for k in range(_RPT // _W):
        pltpu.sync_copy(agg_sh.at[pl.ds(r0 + k * _W, _W)], rows_v.at[0])
        pltpu.sync_copy(rows_v.at[0], out_hbm.at[c, pl.ds(r0 + k * _W, _W)])


@jax.jit
def _segsum(h, edges):
    mesh = plsc.VectorSubcoreMesh(core_axis_name="c", subcore_axis_name="s")
    kfn = pl.kernel(
        _segsum_body,
        out_type=jax.ShapeDtypeStruct((_NC, _NPAD, _NHID), jnp.float32),
        mesh=mesh,
        scratch_types=[
            pltpu.VMEM((_WPW, _W), jnp.int32),
            pltpu.VMEM((_WPW, _W), jnp.int32),
            pltpu.VMEM((2 * _K, _W, _NHID), jnp.float32),
            pltpu.VMEM_SHARED((_NPAD, _NHID), jnp.float32),
            pltpu.SemaphoreType.DMA,
            pltpu.SemaphoreType.DMA,
            pltpu.SemaphoreType.DMA,
            pltpu.SemaphoreType.DMA,
        ],
        compiler_params=pltpu.CompilerParams(use_tc_tiling_on_sc=False),
    )
    return kfn(h, edges)


def _mix(v):
    return v + jnp.concatenate([v[_NHID:], v[:_NHID]])


def _bn_packed(m, g2, be2):
    mu = _mix(jnp.sum(m, axis=0)) / _N
    var = _mix(jnp.sum((m - mu) ** 2, axis=0)) / _N
    return g2 * (m - mu) * lax.rsqrt(var + _EPS) + be2


def _pool_packed(hp, oh_ref, cntinv_ref):
    dn = (((0,), (0,)), ((), ()))
    s = lax.dot_general(oh_ref[:, :_G], hp[:, :_NHID], dn,
                        preferred_element_type=jnp.float32)
    s = s + lax.dot_general(oh_ref[:, _G:], hp[:, _NHID:], dn,
                            preferred_element_type=jnp.float32)
    return s * cntinv_ref[...]


def _prologue_body(x_ref, w_ref, b_ref, g_ref, be_ref, oh_ref, cntinv_ref,
                   h_ref, emb_ref):
    h = jnp.dot(x_ref[...], w_ref[...], preferred_element_type=jnp.float32)
    h = _bn_packed(h + b_ref[...], g_ref[...], be_ref[...])
    h_ref[...] = h
    emb_ref[...] = _pool_packed(h, oh_ref, cntinv_ref)


@jax.jit
def _prologue(xp, w, b, g, be, oh, cntinv):
    return pl.pallas_call(
        _prologue_body,
        out_shape=(jax.ShapeDtypeStruct((_NP2, 2 * _NHID), jnp.float32),
                   jax.ShapeDtypeStruct((_G, _NHID), jnp.float32)),
    )(xp, w, b, g, be, oh, cntinv)


def _layer_body(h_ref, agg_ref, wa_ref, ba_ref, wb_ref, bb_ref,
                g_ref, be_ref, oh_ref, cntinv_ref, ho_ref, emb_ref):
    z = h_ref[...] + agg_ref[0, :_NP2, :] + agg_ref[1, :_NP2, :]
    m = jnp.dot(z, wa_ref[...], preferred_element_type=jnp.float32)
    m = jnp.maximum(m + ba_ref[...], 0.0)
    m = jnp.dot(m, wb_ref[...], preferred_element_type=jnp.float32)
    m = m + bb_ref[...]
    h = jnp.maximum(_bn_packed(m, g_ref[...], be_ref[...]), 0.0)
    ho_ref[...] = h
    emb_ref[...] = _pool_packed(h, oh_ref, cntinv_ref)


@jax.jit
def _layer(hp, aggs, wa, ba, wb, bb, g, be, oh, cntinv):
    return pl.pallas_call(
        _layer_body,
        out_shape=(jax.ShapeDtypeStruct((_NP2, 2 * _NHID), jnp.float32),
                   jax.ShapeDtypeStruct((_G, _NHID), jnp.float32)),
    )(hp, aggs, wa, ba, wb, bb, g, be, oh, cntinv)


def _blockdiag(w):
    k, n = w.shape
    z = jnp.zeros((k, n), jnp.float32)
    return jnp.concatenate([jnp.concatenate([w, z], axis=1),
                            jnp.concatenate([z, w], axis=1)], axis=0)


def _tile2(v):
    return jnp.concatenate([v, v])[None, :]


def kernel(x, params, edge_index, batch):
    npad_e = _EPAD - _E
    pad_iota = jnp.arange(npad_e, dtype=jnp.int32)
    pad = jnp.stack([pad_iota % _N, _N + pad_iota % (_NPAD - _N)])
    edges = jnp.concatenate([edge_index, pad], axis=1).reshape(2, _NWIN, _W)

    onehot = (batch[:, None] == jnp.arange(_G, dtype=jnp.int32)[None, :])
    onehot = onehot.astype(jnp.float32)
    cntinv = 1.0 / jnp.maximum(jnp.sum(onehot, axis=0), 1.0)[:, None]
    oh = onehot.reshape(_NP2, 2 * _G)

    p = params
    xp = x.reshape(_NP2, 2 * _NFEAT)
    hp, emb = _prologue(xp, _blockdiag(p['W0']), _tile2(p['b0']),
                        _tile2(p['g0']), _tile2(p['be0']), oh, cntinv)
    embeds = [emb]
    for L in p['layers']:
        h64 = hp.reshape(_N, _NHID)
        aggs = _segsum(h64, edges)
        aggs_p = aggs.reshape(_NC, _NPAD // 2, 2 * _NHID)
        hp, emb = _layer(hp, aggs_p, _blockdiag(L['Wa']), _tile2(L['ba']),
                         _blockdiag(L['Wb']), _tile2(L['bb']),
                         _tile2(L['g']), _tile2(L['be']), oh, cntinv)
        embeds.append(emb)
    return jnp.stack(embeds)
